# baseline (speedup 1.0000x reference)
import jax
import jax.numpy as jnp
from jax.experimental import pallas as pl
from jax.experimental.pallas import tpu as pltpu

_E = 8
_TB = 128
_BI = 512
_TBR = 512
_TBS = 256
_VMEM = 134217728


def _router_body(hs_ref, rw_ref, out_ref):
    x = hs_ref[...]
    rw = rw_ref[...]
    logits = jax.lax.dot_general(
        rw, x, (((1,), (1,)), ((), ())),
        preferred_element_type=jnp.float32)
    top_v = jnp.max(logits, axis=0, keepdims=True)
    top_i = jnp.argmax(logits, axis=0).reshape(1, -1).astype(jnp.int32)
    sig = jax.nn.sigmoid(top_v)
    eids = jax.lax.broadcasted_iota(jnp.int32, out_ref.shape, 0)
    out_ref[...] = jnp.where(eids == top_i, sig, 0.0)


def _grouped_act_body(blk_ref, exp_ref, first_ref, offs_ref,
                      x_ref, wg_ref, wu_ref, act_ref):
    s = pl.program_id(1)
    x = x_ref[...]
    gate = jnp.dot(x, wg_ref[0].astype(jnp.bfloat16),
                   preferred_element_type=jnp.float32)
    up = jnp.dot(x, wu_ref[0].astype(jnp.bfloat16),
                 preferred_element_type=jnp.float32)
    act = up * (gate * jax.nn.sigmoid(gate))
    e = exp_ref[s]
    lo = offs_ref[e]
    hi = offs_ref[e + 1]
    rows = jax.lax.broadcasted_iota(jnp.int32, act.shape, 0) + blk_ref[s] * _TB
    act = jnp.where((rows >= lo) & (rows < hi), act, 0.0).astype(jnp.bfloat16)

    @pl.when(first_ref[s] == 1)
    def _():
        act_ref[...] = act

    @pl.when(first_ref[s] != 1)
    def _():
        act_ref[...] += act


def _grouped_down_body(blk_ref, exp_ref, first_ref, offs_ref,
                       act_ref, wd_ref, o_ref):
    s = pl.program_id(0)
    e = exp_ref[s]
    lo = offs_ref[e]
    hi = offs_ref[e + 1]
    act = act_ref[...].astype(jnp.float32)
    rows = jax.lax.broadcasted_iota(jnp.int32, act.shape, 0) + blk_ref[s] * _TB
    act = jnp.where((rows >= lo) & (rows < hi), act, 0.0)
    contrib = jnp.dot(act, wd_ref[0], preferred_element_type=jnp.float32)

    @pl.when(first_ref[s] == 1)
    def _():
        o_ref[...] = contrib

    @pl.when(first_ref[s] != 1)
    def _():
        o_ref[...] += contrib


_DN_T = (((1,), (1,)), ((), ()))


def _shared_act_body(hs_ref, wg_ref, wu_ref, act_ref, wg_bf, wu_bf):
    @pl.when(pl.program_id(0) == 0)
    def _():
        wg_bf[...] = wg_ref[...].astype(jnp.bfloat16)
        wu_bf[...] = wu_ref[...].astype(jnp.bfloat16)

    x = hs_ref[...].astype(jnp.bfloat16)
    gate = jax.lax.dot_general(x, wg_bf[...], _DN_T,
                               preferred_element_type=jnp.float32)
    up = jax.lax.dot_general(x, wu_bf[...], _DN_T,
                             preferred_element_type=jnp.float32)
    act_ref[...] = (up * (gate * jax.nn.sigmoid(gate))).astype(jnp.bfloat16)


def _shared_down_body(act_ref, wd_ref, r_ref, o_ref, wd_bf):
    @pl.when(pl.program_id(0) == 0)
    def _():
        wd_bf[...] = wd_ref[...].astype(jnp.bfloat16)

    o_ref[...] = r_ref[...] + jax.lax.dot_general(
        act_ref[...], wd_bf[...], _DN_T, preferred_element_type=jnp.float32)


def kernel(hidden_states, router_w, gate_up_proj, down_proj, gate_w, up_w, down_w):
    b, s_len, h = hidden_states.shape
    hs = hidden_states.reshape(-1, h)
    t = hs.shape[0]
    i_dim = down_proj.shape[1]
    nb = t // _TB
    nib = i_dim // _BI
    n_steps = nb + _E - 1

    router_scores = pl.pallas_call(
        _router_body,
        grid=(t // _TBR,),
        in_specs=[
            pl.BlockSpec((_TBR, h), lambda i: (i, 0)),
            pl.BlockSpec((_E, h), lambda i: (0, 0)),
        ],
        out_specs=pl.BlockSpec((_E, _TBR), lambda i: (0, i)),
        out_shape=jax.ShapeDtypeStruct((_E, t), jnp.float32),
    )(hs, router_w)

    top_i = jnp.argmax(router_scores, axis=0).astype(jnp.int32)
    score_t = jnp.max(router_scores, axis=0)
    sort_idx = jnp.argsort(top_i)
    sizes = jnp.bincount(top_i, length=_E).astype(jnp.int32)
    offs = jnp.concatenate(
        [jnp.zeros((1,), jnp.int32), jnp.cumsum(sizes).astype(jnp.int32)])
    offs_pad = jnp.concatenate([offs, jnp.full((1,), t, jnp.int32)])

    start_blk = offs[:_E] // _TB
    end_blk = (offs[1:_E + 1] - 1) // _TB
    nblk = jnp.where(sizes > 0, end_blk - start_blk + 1, 0)
    step_start = jnp.cumsum(nblk) - nblk
    num_real = jnp.sum(nblk)
    sidx = jnp.arange(n_steps, dtype=jnp.int32)
    expert_of = jnp.repeat(
        jnp.arange(_E, dtype=jnp.int32), nblk, total_repeat_length=n_steps)
    valid = sidx < num_real
    block_id = jnp.where(
        valid, start_blk[expert_of] + (sidx - step_start[expert_of]), nb - 1
    ).astype(jnp.int32)
    expert_id = jnp.where(valid, expert_of, _E).astype(jnp.int32)
    prev_blk = jnp.concatenate([jnp.full((1,), -1, jnp.int32), block_id[:-1]])
    first_flag = (block_id != prev_blk).astype(jnp.int32)

    x_sorted = (hs[sort_idx] * score_t[sort_idx][:, None]).astype(jnp.bfloat16)

    def _wexp(exp):
        return jnp.minimum(exp, _E - 1)

    act_sorted = pl.pallas_call(
        _grouped_act_body,
        grid_spec=pltpu.PrefetchScalarGridSpec(
            num_scalar_prefetch=4,
            grid=(nib, n_steps),
            in_specs=[
                pl.BlockSpec((_TB, h),
                             lambda ib, s, blk, exp, fst, off: (blk[s], 0)),
                pl.BlockSpec((1, h, _BI),
                             lambda ib, s, blk, exp, fst, off: (_wexp(exp[s]), 0, ib)),
                pl.BlockSpec((1, h, _BI),
                             lambda ib, s, blk, exp, fst, off:
                             (_wexp(exp[s]), 0, ib + (down_proj.shape[1] // _BI))),
            ],
            out_specs=pl.BlockSpec((_TB, _BI),
                                   lambda ib, s, blk, exp, fst, off: (blk[s], ib)),
        ),
        out_shape=jax.ShapeDtypeStruct((t, i_dim), jnp.bfloat16),
        compiler_params=pltpu.CompilerParams(
            dimension_semantics=("arbitrary", "arbitrary"),
            vmem_limit_bytes=_VMEM),
    )(block_id, expert_id, first_flag, offs_pad,
      x_sorted, gate_up_proj, gate_up_proj)

    y_sorted = pl.pallas_call(
        _grouped_down_body,
        grid_spec=pltpu.PrefetchScalarGridSpec(
            num_scalar_prefetch=4,
            grid=(n_steps,),
            in_specs=[
                pl.BlockSpec((_TB, i_dim),
                             lambda s, blk, exp, fst, off: (blk[s], 0)),
                pl.BlockSpec((1, i_dim, h),
                             lambda s, blk, exp, fst, off: (_wexp(exp[s]), 0, 0)),
            ],
            out_specs=pl.BlockSpec((_TB, h),
                                   lambda s, blk, exp, fst, off: (blk[s], 0)),
        ),
        out_shape=jax.ShapeDtypeStruct((t, h), jnp.float32),
        compiler_params=pltpu.CompilerParams(
            dimension_semantics=("arbitrary",),
            vmem_limit_bytes=_VMEM),
    )(block_id, expert_id, first_flag, offs_pad, act_sorted, down_proj)

    routed_nat = jnp.zeros((t, h), jnp.float32).at[sort_idx].set(y_sorted)

    act = pl.pallas_call(
        _shared_act_body,
        grid=(t // _TBS,),
        in_specs=[
            pl.BlockSpec((_TBS, h), lambda i: (i, 0)),
            pl.BlockSpec((i_dim, h), lambda i: (0, 0)),
            pl.BlockSpec((i_dim, h), lambda i: (0, 0)),
        ],
        out_specs=pl.BlockSpec((_TBS, i_dim), lambda i: (i, 0)),
        out_shape=jax.ShapeDtypeStruct((t, i_dim), jnp.bfloat16),
        scratch_shapes=[
            pltpu.VMEM((i_dim, h), jnp.bfloat16),
            pltpu.VMEM((i_dim, h), jnp.bfloat16),
        ],
        compiler_params=pltpu.CompilerParams(
            dimension_semantics=("arbitrary",),
            vmem_limit_bytes=_VMEM),
    )(hs, gate_w, up_w)

    out = pl.pallas_call(
        _shared_down_body,
        grid=(t // _TBS,),
        in_specs=[
            pl.BlockSpec((_TBS, i_dim), lambda i: (i, 0)),
            pl.BlockSpec((h, i_dim), lambda i: (0, 0)),
            pl.BlockSpec((_TBS, h), lambda i: (i, 0)),
        ],
        out_specs=pl.BlockSpec((_TBS, h), lambda i: (i, 0)),
        out_shape=jax.ShapeDtypeStruct((t, h), jnp.float32),
        scratch_shapes=[
            pltpu.VMEM((h, i_dim), jnp.bfloat16),
        ],
        compiler_params=pltpu.CompilerParams(
            dimension_semantics=("arbitrary",),
            vmem_limit_bytes=_VMEM),
    )(act, down_w, routed_nat)

    return out, router_scores


# speedup vs baseline: 1.0338x; 1.0338x over previous
import jax
import jax.numpy as jnp
from jax.experimental import pallas as pl
from jax.experimental.pallas import tpu as pltpu

_E = 8
_TB = 256
_BI = 512
_TBR = 512
_TBS = 256
_VMEM = 134217728


def _router_body(hs_ref, rw_ref, out_ref):
    x = hs_ref[...]
    rw = rw_ref[...]
    logits = jax.lax.dot_general(
        rw, x, (((1,), (1,)), ((), ())),
        preferred_element_type=jnp.float32)
    top_v = jnp.max(logits, axis=0, keepdims=True)
    top_i = jnp.argmax(logits, axis=0).reshape(1, -1).astype(jnp.int32)
    sig = jax.nn.sigmoid(top_v)
    eids = jax.lax.broadcasted_iota(jnp.int32, out_ref.shape, 0)
    out_ref[...] = jnp.where(eids == top_i, sig, 0.0)


def _grouped_body(blk_ref, exp_ref, first_ref, offs_ref,
                  x_ref, wg_ref, wu_ref, wd_ref, o_ref):
    s = pl.program_id(0)
    ib = pl.program_id(1)
    x = x_ref[...]
    gate = jnp.dot(x, wg_ref[0].astype(jnp.bfloat16),
                   preferred_element_type=jnp.float32)
    up = jnp.dot(x, wu_ref[0].astype(jnp.bfloat16),
                 preferred_element_type=jnp.float32)
    act = up * (gate * jax.nn.sigmoid(gate))
    e = exp_ref[s]
    lo = offs_ref[e]
    hi = offs_ref[e + 1]
    rows = jax.lax.broadcasted_iota(jnp.int32, act.shape, 0) + blk_ref[s] * _TB
    act = jnp.where((rows >= lo) & (rows < hi), act, 0.0).astype(jnp.bfloat16)
    contrib = jnp.dot(act, wd_ref[0].astype(jnp.bfloat16),
                      preferred_element_type=jnp.float32)
    first = (first_ref[s] == 1) & (ib == 0)

    @pl.when(first)
    def _():
        o_ref[...] = contrib

    @pl.when(jnp.logical_not(first))
    def _():
        o_ref[...] += contrib


_DN_T = (((1,), (1,)), ((), ()))


def _shared_act_body(hs_ref, wg_ref, wu_ref, act_ref, wg_bf, wu_bf):
    @pl.when(pl.program_id(0) == 0)
    def _():
        wg_bf[...] = wg_ref[...].astype(jnp.bfloat16)
        wu_bf[...] = wu_ref[...].astype(jnp.bfloat16)

    x = hs_ref[...].astype(jnp.bfloat16)
    gate = jax.lax.dot_general(x, wg_bf[...], _DN_T,
                               preferred_element_type=jnp.float32)
    up = jax.lax.dot_general(x, wu_bf[...], _DN_T,
                             preferred_element_type=jnp.float32)
    act_ref[...] = (up * (gate * jax.nn.sigmoid(gate))).astype(jnp.bfloat16)


def _shared_down_body(act_ref, wd_ref, r_ref, o_ref, wd_bf):
    @pl.when(pl.program_id(0) == 0)
    def _():
        wd_bf[...] = wd_ref[...].astype(jnp.bfloat16)

    o_ref[...] = r_ref[...] + jax.lax.dot_general(
        act_ref[...], wd_bf[...], _DN_T, preferred_element_type=jnp.float32)


def kernel(hidden_states, router_w, gate_up_proj, down_proj, gate_w, up_w, down_w):
    b, s_len, h = hidden_states.shape
    hs = hidden_states.reshape(-1, h)
    t = hs.shape[0]
    i_dim = down_proj.shape[1]
    nb = t // _TB
    nib = i_dim // _BI
    n_steps = nb + _E - 1

    router_scores = pl.pallas_call(
        _router_body,
        grid=(t // _TBR,),
        in_specs=[
            pl.BlockSpec((_TBR, h), lambda i: (i, 0)),
            pl.BlockSpec((_E, h), lambda i: (0, 0)),
        ],
        out_specs=pl.BlockSpec((_E, _TBR), lambda i: (0, i)),
        out_shape=jax.ShapeDtypeStruct((_E, t), jnp.float32),
    )(hs, router_w)

    top_i = jnp.argmax(router_scores, axis=0).astype(jnp.int32)
    score_t = jnp.max(router_scores, axis=0)
    sort_idx = jnp.argsort(top_i)
    sizes = jnp.bincount(top_i, length=_E).astype(jnp.int32)
    offs = jnp.concatenate(
        [jnp.zeros((1,), jnp.int32), jnp.cumsum(sizes).astype(jnp.int32)])
    offs_pad = jnp.concatenate([offs, jnp.full((1,), t, jnp.int32)])

    start_blk = offs[:_E] // _TB
    end_blk = (offs[1:_E + 1] - 1) // _TB
    nblk = jnp.where(sizes > 0, end_blk - start_blk + 1, 0)
    step_start = jnp.cumsum(nblk) - nblk
    num_real = jnp.sum(nblk)
    sidx = jnp.arange(n_steps, dtype=jnp.int32)
    expert_of = jnp.repeat(
        jnp.arange(_E, dtype=jnp.int32), nblk, total_repeat_length=n_steps)
    valid = sidx < num_real
    block_id = jnp.where(
        valid, start_blk[expert_of] + (sidx - step_start[expert_of]), nb - 1
    ).astype(jnp.int32)
    expert_id = jnp.where(valid, expert_of, _E).astype(jnp.int32)
    prev_blk = jnp.concatenate([jnp.full((1,), -1, jnp.int32), block_id[:-1]])
    first_flag = (block_id != prev_blk).astype(jnp.int32)

    x_sorted = (hs[sort_idx] * score_t[sort_idx][:, None]).astype(jnp.bfloat16)

    def _wexp(exp):
        return jnp.minimum(exp, _E - 1)

    y_sorted = pl.pallas_call(
        _grouped_body,
        grid_spec=pltpu.PrefetchScalarGridSpec(
            num_scalar_prefetch=4,
            grid=(n_steps, nib),
            in_specs=[
                pl.BlockSpec((_TB, h),
                             lambda s, ib, blk, exp, fst, off: (blk[s], 0)),
                pl.BlockSpec((1, h, _BI),
                             lambda s, ib, blk, exp, fst, off: (_wexp(exp[s]), 0, ib)),
                pl.BlockSpec((1, h, _BI),
                             lambda s, ib, blk, exp, fst, off:
                             (_wexp(exp[s]), 0, ib + (down_proj.shape[1] // _BI))),
                pl.BlockSpec((1, _BI, h),
                             lambda s, ib, blk, exp, fst, off: (_wexp(exp[s]), ib, 0)),
            ],
            out_specs=pl.BlockSpec((_TB, h),
                                   lambda s, ib, blk, exp, fst, off: (blk[s], 0)),
        ),
        out_shape=jax.ShapeDtypeStruct((t, h), jnp.float32),
        compiler_params=pltpu.CompilerParams(
            dimension_semantics=("arbitrary", "arbitrary"),
            vmem_limit_bytes=_VMEM),
    )(block_id, expert_id, first_flag, offs_pad,
      x_sorted, gate_up_proj, gate_up_proj, down_proj)

    routed_nat = jnp.zeros((t, h), jnp.float32).at[sort_idx].set(y_sorted)

    act = pl.pallas_call(
        _shared_act_body,
        grid=(t // _TBS,),
        in_specs=[
            pl.BlockSpec((_TBS, h), lambda i: (i, 0)),
            pl.BlockSpec((i_dim, h), lambda i: (0, 0)),
            pl.BlockSpec((i_dim, h), lambda i: (0, 0)),
        ],
        out_specs=pl.BlockSpec((_TBS, i_dim), lambda i: (i, 0)),
        out_shape=jax.ShapeDtypeStruct((t, i_dim), jnp.bfloat16),
        scratch_shapes=[
            pltpu.VMEM((i_dim, h), jnp.bfloat16),
            pltpu.VMEM((i_dim, h), jnp.bfloat16),
        ],
        compiler_params=pltpu.CompilerParams(
            dimension_semantics=("arbitrary",),
            vmem_limit_bytes=_VMEM),
    )(hs, gate_w, up_w)

    out = pl.pallas_call(
        _shared_down_body,
        grid=(t // _TBS,),
        in_specs=[
            pl.BlockSpec((_TBS, i_dim), lambda i: (i, 0)),
            pl.BlockSpec((h, i_dim), lambda i: (0, 0)),
            pl.BlockSpec((_TBS, h), lambda i: (i, 0)),
        ],
        out_specs=pl.BlockSpec((_TBS, h), lambda i: (i, 0)),
        out_shape=jax.ShapeDtypeStruct((t, h), jnp.float32),
        scratch_shapes=[
            pltpu.VMEM((h, i_dim), jnp.bfloat16),
        ],
        compiler_params=pltpu.CompilerParams(
            dimension_semantics=("arbitrary",),
            vmem_limit_bytes=_VMEM),
    )(act, down_w, routed_nat)

    return out, router_scores
